# baseline (speedup 1.0000x reference)
import functools

import jax
import jax.numpy as jnp
from jax import lax
from jax.experimental import pallas as pl
from jax.experimental.pallas import tpu as pltpu
from jax.experimental.pallas import tpu_sc as plsc

_THRESH = 3000.0
_HIGHEST = jax.lax.Precision.HIGHEST

_NC, _NS, _L = 2, 16, 16
_NW = _NC * _NS
_ROWS = 64
_W = 512
_STRIPE = _ROWS * _W


def _lane_sum(vec):
    s = [vec[i] for i in range(_L)]
    while len(s) > 1:
        s = [a + b for a, b in zip(s[::2], s[1::2])]
    return s[0]


def _sc_stats_body(x_hbm, stats_hbm, xbuf, statv, sem):
    c = lax.axis_index("c")
    s = lax.axis_index("s")
    w = c * _NS + s
    base = w * _STRIPE

    cp = pltpu.make_async_copy(x_hbm.at[pl.ds(base, _STRIPE)], xbuf, sem)
    cp.start()
    cp.wait()

    zero = jnp.zeros((_L,), jnp.float32)
    lane = lax.iota(jnp.int32, _L)
    statvec = zero
    for j in range(8):
        col = j * 64

        def sum_row(r, acc, col=col):
            o = r * _W + col
            a = xbuf[pl.ds(o, _L)] + xbuf[pl.ds(o + 16, _L)]
            b = xbuf[pl.ds(o + 32, _L)] + xbuf[pl.ds(o + 48, _L)]
            return acc + (a + b)

        ssum = _lane_sum(lax.fori_loop(0, _ROWS, sum_row, zero))
        mv = jnp.full((_L,), ssum * (1.0 / 4096.0), jnp.float32)

        def var_row(r, acc, col=col, mv=mv):
            o = r * _W + col
            d0 = xbuf[pl.ds(o, _L)] - mv
            d1 = xbuf[pl.ds(o + 16, _L)] - mv
            d2 = xbuf[pl.ds(o + 32, _L)] - mv
            d3 = xbuf[pl.ds(o + 48, _L)] - mv
            return acc + ((d0 * d0 + d1 * d1) + (d2 * d2 + d3 * d3))

        vsum = _lane_sum(lax.fori_loop(0, _ROWS, var_row, zero))
        statvec = jnp.where(lane == j, ssum, statvec)
        statvec = jnp.where(lane == 8 + j, vsum, statvec)

    statv[...] = statvec
    pltpu.sync_copy(statv, stats_hbm.at[pl.ds(w * _L, _L)])


def _sc_stats(x1d):
    mesh = plsc.VectorSubcoreMesh(
        core_axis_name="c", subcore_axis_name="s",
        num_cores=_NC, num_subcores=_NS,
    )
    return pl.kernel(
        _sc_stats_body,
        out_type=jax.ShapeDtypeStruct((_NW * _L,), jnp.float32),
        mesh=mesh,
        scratch_types=[
            pltpu.VMEM((_STRIPE,), jnp.float32),
            pltpu.VMEM((_L,), jnp.float32),
            pltpu.SemaphoreType.DMA,
        ],
    )(x1d)


def _block_mat(n, m):
    r = lax.broadcasted_iota(jnp.int32, (n, m), 0)
    c = lax.broadcasted_iota(jnp.int32, (n, m), 1)
    return (r // (n // m) == c).astype(jnp.float32)


def _block_mat_t(m, n):
    r = lax.broadcasted_iota(jnp.int32, (m, n), 0)
    c = lax.broadcasted_iota(jnp.int32, (m, n), 1)
    return (c // (n // m) == r).astype(jnp.float32)


def _dot(a, b):
    return lax.dot(a, b, precision=_HIGHEST, preferred_element_type=jnp.float32)


def _tc_render_body(thr_ref, s64_ref, v64_ref, o_ref):
    s64 = s64_ref[0]
    v64 = v64_ref[0]
    m64 = s64 * (1.0 / 4096.0)

    p = _block_mat(512, 8)
    pt = _block_mat_t(8, 512)
    u = _block_mat(8, 4)
    ut = _block_mat_t(4, 8)
    e = _block_mat(8, 2)
    et = _block_mat_t(2, 8)
    w = _block_mat(4, 2)
    wt = _block_mat_t(2, 4)

    m128 = _dot(_dot(ut, m64), u) * 0.25
    m128e = _dot(_dot(u, m128), ut)
    dm = m64 - m128e
    v128 = _dot(_dot(ut, v64 + 4096.0 * dm * dm), u)

    m256 = _dot(_dot(wt, m128), w) * 0.25
    m256e4 = _dot(_dot(w, m256), wt)
    dm = m128 - m256e4
    v256 = _dot(_dot(wt, v128 + 16384.0 * dm * dm), w)

    m512 = jnp.sum(m256) * 0.25
    dm = m256 - m512
    v512 = jnp.sum(v256) + 65536.0 * jnp.sum(dm * dm)

    s0 = (v512 >= thr_ref[0]).astype(jnp.float32)
    s1 = (v256 >= thr_ref[1]).astype(jnp.float32)
    s2 = (v128 >= thr_ref[2]).astype(jnp.float32)

    s1e = _dot(_dot(e, s1), et)
    s2e = _dot(_dot(u, s2), ut)
    m256e = _dot(_dot(e, m256), et)
    m128e8 = _dot(_dot(u, m128), ut)

    inner = (1.0 - s2e) * m128e8 + s2e * m64
    mid = (1.0 - s1e) * m256e + s1e * inner
    value = (1.0 - s0) * m512 + s0 * mid

    o_ref[0] = _dot(_dot(p, value), pt)


def _tc_render(thr, s64, v64, b):
    return pl.pallas_call(
        _tc_render_body,
        grid=(b,),
        in_specs=[
            pl.BlockSpec(memory_space=pltpu.SMEM),
            pl.BlockSpec((1, 8, 8), lambda i: (i, 0, 0)),
            pl.BlockSpec((1, 8, 8), lambda i: (i, 0, 0)),
        ],
        out_specs=pl.BlockSpec((1, _W, _W), lambda i: (i, 0, 0)),
        out_shape=jax.ShapeDtypeStruct((b, _W, _W), jnp.float32),
    )(thr, s64, v64)


def kernel(x, level):
    b, ch, h, w = x.shape
    stats = _sc_stats(x.reshape(b * ch * h * w))
    st = stats.reshape(b, 8, 2, 8)
    s64 = st[:, :, 0, :]
    v64 = st[:, :, 1, :]

    ns = jnp.array([262144.0, 65536.0, 16384.0], dtype=jnp.float32)
    thr = jnp.where(
        jnp.arange(3) == level,
        jnp.float32(jnp.inf),
        (_THRESH * _THRESH) * (ns - 1.0),
    ).astype(jnp.float32)

    out = _tc_render(thr, s64, v64, b)
    return out.reshape(b, ch, h, w)


# speedup vs baseline: 1.1052x; 1.1052x over previous
import jax
import jax.numpy as jnp
from jax import lax
from jax.experimental import pallas as pl
from jax.experimental.pallas import tpu as pltpu
from jax.experimental.pallas import tpu_sc as plsc

_THRESH = 3000.0

_NC, _NS, _L = 2, 16, 16
_ROWS = 64
_W = 512
_STRIPE = _ROWS * _W


def _lane_sum(vec):
    s = [vec[i] for i in range(_L)]
    while len(s) > 1:
        s = [a + b for a, b in zip(s[::2], s[1::2])]
    return s[0]


def _qt_body(x_hbm, thr_hbm, out_hbm, xbuf, statv, allst, thrv, shared, sem):
    c = lax.axis_index("c")
    s = lax.axis_index("s")
    img_local = s // 8
    stripe = s % 8
    g = (c * _NC + img_local) * 8 + stripe
    base = g * _STRIPE

    cp = pltpu.make_async_copy(x_hbm.at[pl.ds(base, _STRIPE)], xbuf, sem)
    cp.start()
    pltpu.sync_copy(thr_hbm, thrv)
    cp.wait()

    zero = jnp.zeros((_L,), jnp.float32)
    lane = lax.iota(jnp.int32, _L)
    statvec = zero
    for j in range(8):
        col = j * 64

        def sum_row(r, acc, col=col):
            o = r * _W + col
            a = xbuf[pl.ds(o, _L)] + xbuf[pl.ds(o + 16, _L)]
            b = xbuf[pl.ds(o + 32, _L)] + xbuf[pl.ds(o + 48, _L)]
            return acc + (a + b)

        ssum = _lane_sum(lax.fori_loop(0, _ROWS, sum_row, zero))
        mv = jnp.full((_L,), ssum * (1.0 / 4096.0), jnp.float32)

        def var_row(r, acc, col=col, mv=mv):
            o = r * _W + col
            d0 = xbuf[pl.ds(o, _L)] - mv
            d1 = xbuf[pl.ds(o + 16, _L)] - mv
            d2 = xbuf[pl.ds(o + 32, _L)] - mv
            d3 = xbuf[pl.ds(o + 48, _L)] - mv
            return acc + ((d0 * d0 + d1 * d1) + (d2 * d2 + d3 * d3))

        vsum = _lane_sum(lax.fori_loop(0, _ROWS, var_row, zero))
        statvec = jnp.where(lane == j, ssum, statvec)
        statvec = jnp.where(lane == 8 + j, vsum, statvec)

    statv[pl.ds(0, _L)] = statvec
    pltpu.sync_copy(statv, shared.at[s])
    plsc.subcore_barrier()
    pltpu.sync_copy(shared.at[pl.ds(img_local * 8, 8)], allst)

    rows = [allst[i, pl.ds(0, _L)] for i in range(8)]
    m64 = [[rows[i][j] * (1.0 / 4096.0) for j in range(8)] for i in range(8)]
    v64 = [[rows[i][8 + j] for j in range(8)] for i in range(8)]

    m128, v128 = [], []
    for a in range(4):
        m128.append([])
        v128.append([])
        for b in range(4):
            ms = [m64[2 * a + di][2 * b + dj] for di in range(2) for dj in range(2)]
            vs = [v64[2 * a + di][2 * b + dj] for di in range(2) for dj in range(2)]
            m = ((ms[0] + ms[1]) + (ms[2] + ms[3])) * 0.25
            dv = [mm - m for mm in ms]
            v = ((vs[0] + vs[1]) + (vs[2] + vs[3])) + 4096.0 * (
                (dv[0] * dv[0] + dv[1] * dv[1]) + (dv[2] * dv[2] + dv[3] * dv[3]))
            m128[a].append(m)
            v128[a].append(v)

    m256, v256 = [], []
    for a in range(2):
        m256.append([])
        v256.append([])
        for b in range(2):
            ms = [m128[2 * a + di][2 * b + dj] for di in range(2) for dj in range(2)]
            vs = [v128[2 * a + di][2 * b + dj] for di in range(2) for dj in range(2)]
            m = ((ms[0] + ms[1]) + (ms[2] + ms[3])) * 0.25
            dv = [mm - m for mm in ms]
            v = ((vs[0] + vs[1]) + (vs[2] + vs[3])) + 16384.0 * (
                (dv[0] * dv[0] + dv[1] * dv[1]) + (dv[2] * dv[2] + dv[3] * dv[3]))
            m256[a].append(m)
            v256[a].append(v)

    ms = [m256[0][0], m256[0][1], m256[1][0], m256[1][1]]
    vs = [v256[0][0], v256[0][1], v256[1][0], v256[1][1]]
    m512 = ((ms[0] + ms[1]) + (ms[2] + ms[3])) * 0.25
    dv = [mm - m512 for mm in ms]
    v512 = ((vs[0] + vs[1]) + (vs[2] + vs[3])) + 65536.0 * (
        (dv[0] * dv[0] + dv[1] * dv[1]) + (dv[2] * dv[2] + dv[3] * dv[3]))

    tv = thrv[...]
    thr0, thr1, thr2 = tv[0], tv[1], tv[2]
    s0 = v512 >= thr0

    i2 = stripe // 2
    i4 = stripe // 4

    def sel4(table, idx):
        r = table[3]
        for k in (2, 1, 0):
            r = jnp.where(idx == k, table[k], r)
        return r

    m128r = [sel4([m128[a][b] for a in range(4)], i2) for b in range(4)]
    v128r = [sel4([v128[a][b] for a in range(4)], i2) for b in range(4)]
    m256r = [jnp.where(i4 == 0, m256[0][b], m256[1][b]) for b in range(2)]
    v256r = [jnp.where(i4 == 0, v256[0][b], v256[1][b]) for b in range(2)]
    m64r = [statvec[j] * (1.0 / 4096.0) for j in range(8)]

    for j in range(8):
        inner = jnp.where(v128r[j // 2] >= thr2, m64r[j], m128r[j // 2])
        mid = jnp.where(v256r[j // 4] >= thr1, inner, m256r[j // 4])
        val = jnp.where(s0, mid, m512)
        vj = jnp.full((_L,), val, jnp.float32)

        def fill_row(r, carry, col=j * 64, vj=vj):
            o = r * _W + col
            xbuf[pl.ds(o, _L)] = vj
            xbuf[pl.ds(o + 16, _L)] = vj
            xbuf[pl.ds(o + 32, _L)] = vj
            xbuf[pl.ds(o + 48, _L)] = vj
            return carry

        lax.fori_loop(0, _ROWS, fill_row, 0)

    pltpu.sync_copy(xbuf, out_hbm.at[pl.ds(base, _STRIPE)])


def kernel(x, level):
    b, ch, h, w = x.shape
    x1d = x.reshape(b * ch * h * w)

    ns = jnp.full((_L,), 1.0, jnp.float32)
    ns = ns.at[0].set(262144.0).at[1].set(65536.0).at[2].set(16384.0)
    thr = jnp.where(
        jnp.arange(_L) == level,
        jnp.float32(jnp.inf),
        (_THRESH * _THRESH) * (ns - 1.0),
    ).astype(jnp.float32)

    mesh = plsc.VectorSubcoreMesh(
        core_axis_name="c", subcore_axis_name="s",
        num_cores=_NC, num_subcores=_NS,
    )
    out = pl.kernel(
        _qt_body,
        out_type=jax.ShapeDtypeStruct((b * ch * h * w,), jnp.float32),
        mesh=mesh,
        scratch_types=[
            pltpu.VMEM((_STRIPE,), jnp.float32),
            pltpu.VMEM((128,), jnp.float32),
            pltpu.VMEM((8, 128), jnp.float32),
            pltpu.VMEM((_L,), jnp.float32),
            pltpu.VMEM_SHARED((_NS, 128), jnp.float32),
            pltpu.SemaphoreType.DMA,
        ],
    )(x1d, thr)
    return out.reshape(b, ch, h, w)


# speedup vs baseline: 1.3710x; 1.2405x over previous
import jax
import jax.numpy as jnp
from jax import lax
from jax.experimental import pallas as pl
from jax.experimental.pallas import tpu as pltpu
from jax.experimental.pallas import tpu_sc as plsc

_THRESH = 3000.0

_NC, _NS, _L = 2, 16, 16
_ROWS = 64
_W = 512
_STRIPE = _ROWS * _W


def _lane_sum(vec):
    s = [vec[i] for i in range(_L)]
    while len(s) > 1:
        s = [a + b for a, b in zip(s[::2], s[1::2])]
    return s[0]


def _qt_body(x_hbm, thr_hbm, out_hbm, xbuf, statv, allst, thrv, shared, sem):
    c = lax.axis_index("c")
    s = lax.axis_index("s")
    img_local = s // 8
    stripe = s % 8
    g = (c * _NC + img_local) * 8 + stripe
    base = g * _STRIPE

    cp = pltpu.make_async_copy(x_hbm.at[pl.ds(base, _STRIPE)], xbuf, sem)
    cp.start()
    pltpu.sync_copy(thr_hbm, thrv)
    cp.wait()

    zero = jnp.zeros((_L,), jnp.float32)
    lane = lax.iota(jnp.int32, _L)
    statvec = zero
    for j in range(8):
        jbase = (j // 2) * 1024 + (j % 2) * 64

        def sum_row(t, acc, jbase=jbase):
            o = (t // 8) * 4096 + (t % 8) * 128 + jbase
            a = xbuf[pl.ds(o, _L)] + xbuf[pl.ds(o + 16, _L)]
            b = xbuf[pl.ds(o + 32, _L)] + xbuf[pl.ds(o + 48, _L)]
            return acc + (a + b)

        ssum = _lane_sum(lax.fori_loop(0, _ROWS, sum_row, zero))
        mv = jnp.full((_L,), ssum * (1.0 / 4096.0), jnp.float32)

        def var_row(t, acc, jbase=jbase, mv=mv):
            o = (t // 8) * 4096 + (t % 8) * 128 + jbase
            d0 = xbuf[pl.ds(o, _L)] - mv
            d1 = xbuf[pl.ds(o + 16, _L)] - mv
            d2 = xbuf[pl.ds(o + 32, _L)] - mv
            d3 = xbuf[pl.ds(o + 48, _L)] - mv
            return acc + ((d0 * d0 + d1 * d1) + (d2 * d2 + d3 * d3))

        vsum = _lane_sum(lax.fori_loop(0, _ROWS, var_row, zero))
        statvec = jnp.where(lane == j, ssum, statvec)
        statvec = jnp.where(lane == 8 + j, vsum, statvec)

    statv[pl.ds(0, _L)] = statvec
    pltpu.sync_copy(statv, shared.at[s])
    plsc.subcore_barrier()
    pltpu.sync_copy(shared.at[pl.ds(img_local * 8, 8)], allst)

    rows = [allst[i, pl.ds(0, _L)] for i in range(8)]
    m64 = [[rows[i][j] * (1.0 / 4096.0) for j in range(8)] for i in range(8)]
    v64 = [[rows[i][8 + j] for j in range(8)] for i in range(8)]

    m128, v128 = [], []
    for a in range(4):
        m128.append([])
        v128.append([])
        for b in range(4):
            ms = [m64[2 * a + di][2 * b + dj] for di in range(2) for dj in range(2)]
            vs = [v64[2 * a + di][2 * b + dj] for di in range(2) for dj in range(2)]
            m = ((ms[0] + ms[1]) + (ms[2] + ms[3])) * 0.25
            dv = [mm - m for mm in ms]
            v = ((vs[0] + vs[1]) + (vs[2] + vs[3])) + 4096.0 * (
                (dv[0] * dv[0] + dv[1] * dv[1]) + (dv[2] * dv[2] + dv[3] * dv[3]))
            m128[a].append(m)
            v128[a].append(v)

    m256, v256 = [], []
    for a in range(2):
        m256.append([])
        v256.append([])
        for b in range(2):
            ms = [m128[2 * a + di][2 * b + dj] for di in range(2) for dj in range(2)]
            vs = [v128[2 * a + di][2 * b + dj] for di in range(2) for dj in range(2)]
            m = ((ms[0] + ms[1]) + (ms[2] + ms[3])) * 0.25
            dv = [mm - m for mm in ms]
            v = ((vs[0] + vs[1]) + (vs[2] + vs[3])) + 16384.0 * (
                (dv[0] * dv[0] + dv[1] * dv[1]) + (dv[2] * dv[2] + dv[3] * dv[3]))
            m256[a].append(m)
            v256[a].append(v)

    ms = [m256[0][0], m256[0][1], m256[1][0], m256[1][1]]
    vs = [v256[0][0], v256[0][1], v256[1][0], v256[1][1]]
    m512 = ((ms[0] + ms[1]) + (ms[2] + ms[3])) * 0.25
    dv = [mm - m512 for mm in ms]
    v512 = ((vs[0] + vs[1]) + (vs[2] + vs[3])) + 65536.0 * (
        (dv[0] * dv[0] + dv[1] * dv[1]) + (dv[2] * dv[2] + dv[3] * dv[3]))

    tv = thrv[...]
    thr0, thr1, thr2 = tv[0], tv[1], tv[2]
    s0 = v512 >= thr0

    i2 = stripe // 2
    i4 = stripe // 4

    def sel4(table, idx):
        r = table[3]
        for k in (2, 1, 0):
            r = jnp.where(idx == k, table[k], r)
        return r

    m128r = [sel4([m128[a][b] for a in range(4)], i2) for b in range(4)]
    v128r = [sel4([v128[a][b] for a in range(4)], i2) for b in range(4)]
    m256r = [jnp.where(i4 == 0, m256[0][b], m256[1][b]) for b in range(2)]
    v256r = [jnp.where(i4 == 0, v256[0][b], v256[1][b]) for b in range(2)]
    m64r = [statvec[j] * (1.0 / 4096.0) for j in range(8)]

    for j in range(8):
        inner = jnp.where(v128r[j // 2] >= thr2, m64r[j], m128r[j // 2])
        mid = jnp.where(v256r[j // 4] >= thr1, inner, m256r[j // 4])
        val = jnp.where(s0, mid, m512)
        vj = jnp.full((_L,), val, jnp.float32)

        def fill_row(t, carry, jbase=(j // 2) * 1024 + (j % 2) * 64, vj=vj):
            o = (t // 8) * 4096 + (t % 8) * 128 + jbase
            xbuf[pl.ds(o, _L)] = vj
            xbuf[pl.ds(o + 16, _L)] = vj
            xbuf[pl.ds(o + 32, _L)] = vj
            xbuf[pl.ds(o + 48, _L)] = vj
            return carry

        lax.fori_loop(0, _ROWS, fill_row, 0)

    pltpu.sync_copy(xbuf, out_hbm.at[pl.ds(base, _STRIPE)])


def kernel(x, level):
    b, ch, h, w = x.shape
    x1d = (x.reshape(b, h // 8, 8, w // 128, 128)
            .transpose(0, 1, 3, 2, 4)
            .reshape(b * ch * h * w))

    ns = jnp.full((_L,), 1.0, jnp.float32)
    ns = ns.at[0].set(262144.0).at[1].set(65536.0).at[2].set(16384.0)
    thr = jnp.where(
        jnp.arange(_L) == level,
        jnp.float32(jnp.inf),
        (_THRESH * _THRESH) * (ns - 1.0),
    ).astype(jnp.float32)

    mesh = plsc.VectorSubcoreMesh(
        core_axis_name="c", subcore_axis_name="s",
        num_cores=_NC, num_subcores=_NS,
    )
    out = pl.kernel(
        _qt_body,
        out_type=jax.ShapeDtypeStruct((b * ch * h * w,), jnp.float32),
        mesh=mesh,
        scratch_types=[
            pltpu.VMEM((_STRIPE,), jnp.float32),
            pltpu.VMEM((128,), jnp.float32),
            pltpu.VMEM((8, 128), jnp.float32),
            pltpu.VMEM((_L,), jnp.float32),
            pltpu.VMEM_SHARED((_NS, 128), jnp.float32),
            pltpu.SemaphoreType.DMA,
        ],
    )(x1d, thr)
    return (out.reshape(b, h // 8, w // 128, 8, 128)
               .transpose(0, 1, 3, 2, 4)
               .reshape(b, ch, h, w))
